# Initial kernel scaffold; baseline (speedup 1.0000x reference)
import jax
import jax.numpy as jnp
from jax.experimental import pallas as pl


def kernel(x, edge_index, emb_dict, W, attn_l, attn_r, Wlin, blin):
    raise NotImplementedError("write your pallas kernel here")


# speedup vs baseline: 11.2069x; 11.2069x over previous
import functools

import jax
import jax.numpy as jnp
from jax import lax
from jax.experimental import pallas as pl
from jax.experimental.pallas import tpu as pltpu
from jax.experimental.pallas import tpu_sc as plsc

N = 10000
E = 320000
D = 128
H = 3
OUT = 128
ATTN_NEG_SLOPE = 0.2
ACT_NEG_SLOPE = 0.01

NC = 2
NS = 16
LANES = 16

HALF = (H * OUT) // NC
TCB = 1000

KA = 80
EPT_A = E // (NC * NS)
NCH_A = EPT_A // KA
RPT = N // NS
ZRA = 25

KB = 16
SKB = 80
EPT_B = E // NS
NSCH_B = EPT_B // SKB
ZRB = 5


def _tc1_body(x_ref, w_ref, alr_ref, z_ref, elr_ref):
    z = jnp.dot(x_ref[...], w_ref[...], preferred_element_type=jnp.float32, precision=lax.Precision.HIGHEST)
    z_ref[...] = z
    elr_ref[...] = jnp.dot(z, alr_ref[...], preferred_element_type=jnp.float32, precision=lax.Precision.HIGHEST)


def _tc1(x, W, ALR):
    return pl.pallas_call(
        _tc1_body,
        grid=(N // TCB,),
        in_specs=[
            pl.BlockSpec((TCB, D), lambda i: (i, 0)),
            pl.BlockSpec((D, H * OUT), lambda i: (0, 0)),
            pl.BlockSpec((H * OUT, LANES), lambda i: (0, 0)),
        ],
        out_specs=[
            pl.BlockSpec((TCB, H * OUT), lambda i: (i, 0)),
            pl.BlockSpec((TCB, LANES), lambda i: (i, 0)),
        ],
        out_shape=[
            jax.ShapeDtypeStruct((N, H * OUT), jnp.float32),
            jax.ShapeDtypeStruct((N, LANES), jnp.float32),
        ],
    )(x, W, ALR)


def _sca_body(elr, src, dst, w_out, s_out,
              srcb, dstb, elsrc, erdst, wstage, stail, accs, sem):
    c = lax.axis_index("c")
    t = lax.axis_index("s")
    row0 = t * RPT

    for r in range(KA):
        stail[r, pl.ds(0, LANES)] = jnp.zeros((LANES,), jnp.float32)

    def zcopy(j, _):
        pltpu.sync_copy(stail.at[pl.ds(0, ZRA)],
                        accs.at[pl.ds(row0 + j * ZRA, ZRA)])
        return 0
    lax.fori_loop(0, RPT // ZRA, zcopy, 0)
    plsc.subcore_barrier()

    base = c * (NS * EPT_A) + t * EPT_A
    rows0 = lax.iota(jnp.int32, LANES)
    czero = jnp.zeros((LANES,), jnp.int32)

    def chunk(j, _):
        e0 = base + j * KA
        pltpu.sync_copy(src.at[pl.ds(e0, KA)], srcb)
        pltpu.sync_copy(dst.at[pl.ds(e0, KA)], dstb)
        pltpu.async_copy(elr.at[srcb], elsrc, sem).wait()
        pltpu.async_copy(elr.at[dstb], erdst, sem).wait()
        for v in range(KA // LANES):
            rows = rows0 + (v * LANES)
            for h in range(H):
                el = plsc.load_gather(elsrc, [rows, czero + h])
                er = plsc.load_gather(erdst, [rows, czero + (H + h)])
                e = el + er
                e = jnp.where(e >= 0.0, e, e * ATTN_NEG_SLOPE)
                w = jnp.exp(e)
                plsc.store_scatter(wstage, [rows, czero + h], w)
                plsc.store_scatter(stail, [rows, czero + h], w)
        pltpu.sync_copy(wstage, w_out.at[pl.ds(e0, KA)])
        for v in range(KA // LANES):
            dv = dstb[pl.ds(v * LANES, LANES)]
            pltpu.sync_copy(stail.at[pl.ds(v * LANES, LANES)],
                            accs.at[dv], add=True)
        return 0
    lax.fori_loop(0, NCH_A, chunk, 0)
    plsc.subcore_barrier()

    pltpu.sync_copy(accs.at[pl.ds(row0, RPT)],
                    s_out.at[c, pl.ds(row0, RPT)])


def _sc_weights(elr, src, dst):
    mesh = plsc.VectorSubcoreMesh(
        core_axis_name="c", subcore_axis_name="s",
        num_cores=NC, num_subcores=NS)
    kfn = functools.partial(
        pl.kernel,
        mesh=mesh,
        compiler_params=pltpu.CompilerParams(
            use_tc_tiling_on_sc=False, needs_layout_passes=False),
        out_type=[
            jax.ShapeDtypeStruct((E, 4), jnp.float32),
            jax.ShapeDtypeStruct((NC, N, LANES), jnp.float32),
        ],
        scratch_types=[
            pltpu.VMEM((KA,), jnp.int32),
            pltpu.VMEM((KA,), jnp.int32),
            pltpu.VMEM((KA, LANES), jnp.float32),
            pltpu.VMEM((KA, LANES), jnp.float32),
            pltpu.VMEM((KA, 4), jnp.float32),
            pltpu.VMEM((KA, LANES), jnp.float32),
            pltpu.VMEM_SHARED((N, LANES), jnp.float32),
            pltpu.SemaphoreType.DMA,
        ],
    )(_sca_body)
    return kfn(elr, src, dst)


def _scb_body(z2, src, dst, w4, acc_out,
              srcb, dstb, wch, zrows, outr, zbuf, acc, sem0, sem1):
    c = lax.axis_index("c")
    t = lax.axis_index("s")
    row0 = t * RPT
    sems = (sem0, sem1)

    for r in range(ZRB):
        for k in range(HALF // LANES):
            zbuf[r, pl.ds(k * LANES, LANES)] = jnp.zeros((LANES,), jnp.float32)

    def zcopy(j, _):
        pltpu.sync_copy(zbuf, acc.at[pl.ds(row0 + j * ZRB, ZRB)])
        return 0
    lax.fori_loop(0, RPT // ZRB, zcopy, 0)
    plsc.subcore_barrier()

    base = t * EPT_B
    czero = jnp.zeros((LANES,), jnp.int32)
    bnd = 8 - 4 * c

    def gather(ci):
        return pltpu.async_copy(
            z2.at[srcb.at[pl.ds(ci * KB, KB)]],
            zrows.at[ci % 2], sems[ci % 2])

    def superchunk(j, _):
        e0 = base + j * SKB
        pltpu.sync_copy(src.at[pl.ds(e0, SKB)], srcb)
        pltpu.sync_copy(dst.at[pl.ds(e0, SKB)], dstb)
        pltpu.sync_copy(w4.at[pl.ds(e0, SKB)], wch)
        for v in range(SKB // LANES):
            sl = pl.ds(v * LANES, LANES)
            srcb[sl] = srcb[sl] * 2 + c
        descs = {0: gather(0)}
        for ci in range(SKB // KB):
            if ci + 1 < SKB // KB:
                descs[ci + 1] = gather(ci + 1)
            descs[ci].wait()

            def edge(i, _):
                ii = czero + (ci * KB + i)
                wlo = plsc.load_gather(wch, [ii, czero + c])
                whi = plsc.load_gather(wch, [ii, czero + (c + 1)])
                for r in range(HALF // LANES):
                    zv = zrows[ci % 2, i, pl.ds(r * LANES, LANES)]
                    wv = jnp.where(r < bnd, wlo, whi)
                    outr[i, pl.ds(r * LANES, LANES)] = zv * wv
                return 0
            lax.fori_loop(0, KB, edge, 0)
            dv = dstb[pl.ds(ci * KB, KB)]
            pltpu.sync_copy(outr, acc.at[dv], add=True)
        return 0
    lax.fori_loop(0, NSCH_B, superchunk, 0)
    plsc.subcore_barrier()

    pltpu.sync_copy(acc.at[pl.ds(row0, RPT)],
                    acc_out.at[c, pl.ds(row0, RPT)])


def _sc_aggregate(z2, src, dst, w4):
    mesh = plsc.VectorSubcoreMesh(
        core_axis_name="c", subcore_axis_name="s",
        num_cores=NC, num_subcores=NS)
    kfn = functools.partial(
        pl.kernel,
        mesh=mesh,
        compiler_params=pltpu.CompilerParams(
            use_tc_tiling_on_sc=False, needs_layout_passes=False),
        out_type=jax.ShapeDtypeStruct((NC, N, HALF), jnp.float32),
        scratch_types=[
            pltpu.VMEM((SKB,), jnp.int32),
            pltpu.VMEM((SKB,), jnp.int32),
            pltpu.VMEM((SKB, 4), jnp.float32),
            pltpu.VMEM((2, KB, HALF), jnp.float32),
            pltpu.VMEM((KB, HALF), jnp.float32),
            pltpu.VMEM((ZRB, HALF), jnp.float32),
            pltpu.VMEM_SHARED((N, HALF), jnp.float32),
            pltpu.SemaphoreType.DMA,
            pltpu.SemaphoreType.DMA,
        ],
    )(_scb_body)
    return kfn(z2, src, dst, w4)


def _tc2_body(a0_ref, a1_ref, s0_ref, s1_ref, wl_ref, bl_ref, out_ref):
    a0 = a0_ref[...]
    a1 = a1_ref[...]
    s = s0_ref[:, 0:H] + s1_ref[:, 0:H]
    sinv = 1.0 / (s + 1e-9)

    def act(p):
        return jnp.where(p >= 0.0, p, p * ACT_NEG_SLOPE)

    p0 = act(a0[:, 0:128] * sinv[:, 0:1])
    p1a = act(a0[:, 128:192] * sinv[:, 1:2])
    p1b = act(a1[:, 0:64] * sinv[:, 1:2])
    p2 = act(a1[:, 64:192] * sinv[:, 2:3])
    acc = jnp.dot(p0, wl_ref[0:128, :], preferred_element_type=jnp.float32, precision=lax.Precision.HIGHEST)
    acc += jnp.dot(p1a, wl_ref[128:192, :], preferred_element_type=jnp.float32, precision=lax.Precision.HIGHEST)
    acc += jnp.dot(p1b, wl_ref[192:256, :], preferred_element_type=jnp.float32, precision=lax.Precision.HIGHEST)
    acc += jnp.dot(p2, wl_ref[256:384, :], preferred_element_type=jnp.float32, precision=lax.Precision.HIGHEST)
    out_ref[...] = acc + bl_ref[...]


def _tc2(acc0, acc1, s0, s1, Wlin, blin2):
    return pl.pallas_call(
        _tc2_body,
        grid=(N // TCB,),
        in_specs=[
            pl.BlockSpec((TCB, HALF), lambda i: (i, 0)),
            pl.BlockSpec((TCB, HALF), lambda i: (i, 0)),
            pl.BlockSpec((TCB, LANES), lambda i: (i, 0)),
            pl.BlockSpec((TCB, LANES), lambda i: (i, 0)),
            pl.BlockSpec((H * OUT, OUT), lambda i: (0, 0)),
            pl.BlockSpec((1, OUT), lambda i: (0, 0)),
        ],
        out_specs=pl.BlockSpec((TCB, OUT), lambda i: (i, 0)),
        out_shape=jax.ShapeDtypeStruct((N, OUT), jnp.float32),
    )(acc0, acc1, s0, s1, Wlin, blin2)


def kernel(x, edge_index, emb_dict, W, attn_l, attn_r, Wlin, blin):
    src = edge_index[0]
    dst = edge_index[1]

    ALR = jnp.zeros((H * OUT, LANES), jnp.float32)
    for h in range(H):
        ALR = ALR.at[h * OUT:(h + 1) * OUT, h].set(attn_l[h])
        ALR = ALR.at[h * OUT:(h + 1) * OUT, H + h].set(attn_r[h])

    z, elr = _tc1(x, W, ALR)
    z2 = z.reshape(NC * N, HALF)
    w4, s_parts = _sc_weights(elr, src, dst)
    accs = _sc_aggregate(z2, src, dst, w4)
    out = _tc2(accs[0], accs[1], s_parts[0], s_parts[1],
               Wlin, blin.reshape(1, OUT))
    return out


# speedup vs baseline: 11.4489x; 1.0216x over previous
import functools

import jax
import jax.numpy as jnp
from jax import lax
from jax.experimental import pallas as pl
from jax.experimental.pallas import tpu as pltpu
from jax.experimental.pallas import tpu_sc as plsc

N = 10000
E = 320000
D = 128
H = 3
OUT = 128
ATTN_NEG_SLOPE = 0.2
ACT_NEG_SLOPE = 0.01

NC = 2
NS = 16
LANES = 16

HALF = (H * OUT) // NC
TCB = 1000

KA = 80
EPT_A = E // (NC * NS)
NCH_A = EPT_A // KA
RPT = N // NS
ZRA = 25

KB = 16
SKB = 80
EPT_B = E // NS
NSCH_B = EPT_B // SKB
ZRB = 5


def _tc1_body(x_ref, w_ref, alr_ref, z_ref, elr_ref):
    z = jnp.dot(x_ref[...], w_ref[...], preferred_element_type=jnp.float32, precision=lax.Precision.HIGHEST)
    z_ref[...] = z
    elr_ref[...] = jnp.dot(z, alr_ref[...], preferred_element_type=jnp.float32, precision=lax.Precision.HIGHEST)


def _tc1(x, W, ALR):
    return pl.pallas_call(
        _tc1_body,
        grid=(N // TCB,),
        in_specs=[
            pl.BlockSpec((TCB, D), lambda i: (i, 0)),
            pl.BlockSpec((D, H * OUT), lambda i: (0, 0)),
            pl.BlockSpec((H * OUT, LANES), lambda i: (0, 0)),
        ],
        out_specs=[
            pl.BlockSpec((TCB, H * OUT), lambda i: (i, 0)),
            pl.BlockSpec((TCB, LANES), lambda i: (i, 0)),
        ],
        out_shape=[
            jax.ShapeDtypeStruct((N, H * OUT), jnp.float32),
            jax.ShapeDtypeStruct((N, LANES), jnp.float32),
        ],
    )(x, W, ALR)


def _sca_body(elr, src, dst, w_out, s_out,
              srcb, dstb, elsrc, erdst, wstage, stail, accs, sem):
    c = lax.axis_index("c")
    t = lax.axis_index("s")
    row0 = t * RPT

    for r in range(KA):
        stail[r, pl.ds(0, LANES)] = jnp.zeros((LANES,), jnp.float32)

    def zcopy(j, _):
        pltpu.sync_copy(stail.at[pl.ds(0, ZRA)],
                        accs.at[pl.ds(row0 + j * ZRA, ZRA)])
        return 0
    lax.fori_loop(0, RPT // ZRA, zcopy, 0)
    plsc.subcore_barrier()

    base = c * (NS * EPT_A) + t * EPT_A
    rows0 = lax.iota(jnp.int32, LANES)
    czero = jnp.zeros((LANES,), jnp.int32)

    def chunk(j, _):
        e0 = base + j * KA
        pltpu.sync_copy(src.at[pl.ds(e0, KA)], srcb)
        pltpu.sync_copy(dst.at[pl.ds(e0, KA)], dstb)
        pltpu.async_copy(elr.at[srcb], elsrc, sem).wait()
        pltpu.async_copy(elr.at[dstb], erdst, sem).wait()
        for v in range(KA // LANES):
            rows = rows0 + (v * LANES)
            for h in range(H):
                el = plsc.load_gather(elsrc, [rows, czero + h])
                er = plsc.load_gather(erdst, [rows, czero + (H + h)])
                e = el + er
                e = jnp.where(e >= 0.0, e, e * ATTN_NEG_SLOPE)
                w = jnp.exp(e)
                plsc.store_scatter(wstage, [rows, czero + h], w)
                plsc.store_scatter(stail, [rows, czero + h], w)
        pltpu.sync_copy(wstage, w_out.at[pl.ds(e0, KA)])
        for v in range(KA // LANES):
            dv = dstb[pl.ds(v * LANES, LANES)]
            pltpu.sync_copy(stail.at[pl.ds(v * LANES, LANES)],
                            accs.at[dv], add=True)
        return 0
    lax.fori_loop(0, NCH_A, chunk, 0)
    plsc.subcore_barrier()

    pltpu.sync_copy(accs.at[pl.ds(row0, RPT)],
                    s_out.at[c, pl.ds(row0, RPT)])


def _sc_weights(elr, src, dst):
    mesh = plsc.VectorSubcoreMesh(
        core_axis_name="c", subcore_axis_name="s",
        num_cores=NC, num_subcores=NS)
    kfn = functools.partial(
        pl.kernel,
        mesh=mesh,
        compiler_params=pltpu.CompilerParams(
            use_tc_tiling_on_sc=False, needs_layout_passes=False),
        out_type=[
            jax.ShapeDtypeStruct((E, 4), jnp.float32),
            jax.ShapeDtypeStruct((NC, N, LANES), jnp.float32),
        ],
        scratch_types=[
            pltpu.VMEM((KA,), jnp.int32),
            pltpu.VMEM((KA,), jnp.int32),
            pltpu.VMEM((KA, LANES), jnp.float32),
            pltpu.VMEM((KA, LANES), jnp.float32),
            pltpu.VMEM((KA, 4), jnp.float32),
            pltpu.VMEM((KA, LANES), jnp.float32),
            pltpu.VMEM_SHARED((N, LANES), jnp.float32),
            pltpu.SemaphoreType.DMA,
        ],
    )(_sca_body)
    return kfn(elr, src, dst)


def _scb_body(z2, src, dst, w4, acc_out,
              srcb, dstb, wch, zrows, outr, zbuf, acc, sem0, sem1, sems_out):
    c = lax.axis_index("c")
    t = lax.axis_index("s")
    row0 = t * RPT
    sems = (sem0, sem1)

    for r in range(ZRB):
        for k in range(HALF // LANES):
            zbuf[r, pl.ds(k * LANES, LANES)] = jnp.zeros((LANES,), jnp.float32)

    def zcopy(j, _):
        pltpu.sync_copy(zbuf, acc.at[pl.ds(row0 + j * ZRB, ZRB)])
        return 0
    lax.fori_loop(0, RPT // ZRB, zcopy, 0)
    plsc.subcore_barrier()

    def zoutr(r, _):
        for k in range(HALF // LANES):
            outr[r, pl.ds(k * LANES, LANES)] = jnp.zeros((LANES,), jnp.float32)
        return 0
    lax.fori_loop(0, KB, zoutr, 0)
    base = t * EPT_B
    czero = jnp.zeros((LANES,), jnp.int32)
    pltpu.async_copy(outr, acc.at[czero], sems_out, add=True)
    bnd = 8 - 4 * c

    def gather(ci):
        return pltpu.async_copy(
            z2.at[srcb.at[pl.ds(ci * KB, KB)]],
            zrows.at[ci % 2], sems[ci % 2])

    def superchunk(j, _):
        e0 = base + j * SKB
        pltpu.sync_copy(src.at[pl.ds(e0, SKB)], srcb)
        pltpu.sync_copy(dst.at[pl.ds(e0, SKB)], dstb)
        pltpu.sync_copy(w4.at[pl.ds(e0, SKB)], wch)
        for v in range(SKB // LANES):
            sl = pl.ds(v * LANES, LANES)
            srcb[sl] = srcb[sl] * 2 + c
        descs = {0: gather(0)}
        for ci in range(SKB // KB):
            if ci + 1 < SKB // KB:
                descs[ci + 1] = gather(ci + 1)
            descs[ci].wait()

            def edge(i, _):
                ii = czero + (ci * KB + i)
                wlo = plsc.load_gather(wch, [ii, czero + c])
                whi = plsc.load_gather(wch, [ii, czero + (c + 1)])
                for r in range(HALF // LANES):
                    zv = zrows[ci % 2, i, pl.ds(r * LANES, LANES)]
                    wv = jnp.where(r < bnd, wlo, whi)
                    outr[i, pl.ds(r * LANES, LANES)] = zv * wv
                return 0
            dv = dstb[pl.ds(ci * KB, KB)]
            pltpu.make_async_copy(outr, acc.at[dv], sems_out).wait()
            lax.fori_loop(0, KB, edge, 0)
            pltpu.async_copy(outr, acc.at[dv], sems_out, add=True)
        return 0
    lax.fori_loop(0, NSCH_B, superchunk, 0)
    pltpu.make_async_copy(outr, acc.at[czero], sems_out).wait()
    plsc.subcore_barrier()

    pltpu.sync_copy(acc.at[pl.ds(row0, RPT)],
                    acc_out.at[c, pl.ds(row0, RPT)])


def _sc_aggregate(z2, src, dst, w4):
    mesh = plsc.VectorSubcoreMesh(
        core_axis_name="c", subcore_axis_name="s",
        num_cores=NC, num_subcores=NS)
    kfn = functools.partial(
        pl.kernel,
        mesh=mesh,
        compiler_params=pltpu.CompilerParams(
            use_tc_tiling_on_sc=False, needs_layout_passes=False),
        out_type=jax.ShapeDtypeStruct((NC, N, HALF), jnp.float32),
        scratch_types=[
            pltpu.VMEM((SKB,), jnp.int32),
            pltpu.VMEM((SKB,), jnp.int32),
            pltpu.VMEM((SKB, 4), jnp.float32),
            pltpu.VMEM((2, KB, HALF), jnp.float32),
            pltpu.VMEM((KB, HALF), jnp.float32),
            pltpu.VMEM((ZRB, HALF), jnp.float32),
            pltpu.VMEM_SHARED((N, HALF), jnp.float32),
            pltpu.SemaphoreType.DMA,
            pltpu.SemaphoreType.DMA,
            pltpu.SemaphoreType.DMA,
        ],
    )(_scb_body)
    return kfn(z2, src, dst, w4)


def _tc2_body(a0_ref, a1_ref, s0_ref, s1_ref, wl_ref, bl_ref, out_ref):
    a0 = a0_ref[...]
    a1 = a1_ref[...]
    s = s0_ref[:, 0:H] + s1_ref[:, 0:H]
    sinv = 1.0 / (s + 1e-9)

    def act(p):
        return jnp.where(p >= 0.0, p, p * ACT_NEG_SLOPE)

    p0 = act(a0[:, 0:128] * sinv[:, 0:1])
    p1a = act(a0[:, 128:192] * sinv[:, 1:2])
    p1b = act(a1[:, 0:64] * sinv[:, 1:2])
    p2 = act(a1[:, 64:192] * sinv[:, 2:3])
    acc = jnp.dot(p0, wl_ref[0:128, :], preferred_element_type=jnp.float32, precision=lax.Precision.HIGHEST)
    acc += jnp.dot(p1a, wl_ref[128:192, :], preferred_element_type=jnp.float32, precision=lax.Precision.HIGHEST)
    acc += jnp.dot(p1b, wl_ref[192:256, :], preferred_element_type=jnp.float32, precision=lax.Precision.HIGHEST)
    acc += jnp.dot(p2, wl_ref[256:384, :], preferred_element_type=jnp.float32, precision=lax.Precision.HIGHEST)
    out_ref[...] = acc + bl_ref[...]


def _tc2(acc0, acc1, s0, s1, Wlin, blin2):
    return pl.pallas_call(
        _tc2_body,
        grid=(N // TCB,),
        in_specs=[
            pl.BlockSpec((TCB, HALF), lambda i: (i, 0)),
            pl.BlockSpec((TCB, HALF), lambda i: (i, 0)),
            pl.BlockSpec((TCB, LANES), lambda i: (i, 0)),
            pl.BlockSpec((TCB, LANES), lambda i: (i, 0)),
            pl.BlockSpec((H * OUT, OUT), lambda i: (0, 0)),
            pl.BlockSpec((1, OUT), lambda i: (0, 0)),
        ],
        out_specs=pl.BlockSpec((TCB, OUT), lambda i: (i, 0)),
        out_shape=jax.ShapeDtypeStruct((N, OUT), jnp.float32),
    )(acc0, acc1, s0, s1, Wlin, blin2)


def kernel(x, edge_index, emb_dict, W, attn_l, attn_r, Wlin, blin):
    src = edge_index[0]
    dst = edge_index[1]

    ALR = jnp.zeros((H * OUT, LANES), jnp.float32)
    for h in range(H):
        ALR = ALR.at[h * OUT:(h + 1) * OUT, h].set(attn_l[h])
        ALR = ALR.at[h * OUT:(h + 1) * OUT, H + h].set(attn_r[h])

    z, elr = _tc1(x, W, ALR)
    z2 = z.reshape(NC * N, HALF)
    w4, s_parts = _sc_weights(elr, src, dst)
    accs = _sc_aggregate(z2, src, dst, w4)
    out = _tc2(accs[0], accs[1], s_parts[0], s_parts[1],
               Wlin, blin.reshape(1, OUT))
    return out
